# Initial kernel scaffold; baseline (speedup 1.0000x reference)
import jax
import jax.numpy as jnp
from jax.experimental import pallas as pl


def kernel(x, edge_index, edge_attr, u, We1, be1, We2, be2, Wn1, bn1, Wn2, bn2, Wg1, bg1, Wg2, bg2):
    raise NotImplementedError("write your pallas kernel here")


# speedup vs baseline: 3.1382x; 3.1382x over previous
import functools

import jax
import jax.numpy as jnp
from jax import lax
from jax.experimental import pallas as pl
from jax.experimental.pallas import tpu as pltpu
from jax.experimental.pallas import tpu_sc as plsc

E = 1_600_000
N = 50_000
LAT = 64

NC, NS = 2, 16
NW = NC * NS
G = 128
NGRP = E // G
KG = 20
NBLK = NGRP // KG
KMAX = -(-NBLK // NW)
EB = KG * G
KGS = 5
NBLKS = NGRP // KGS
KMAXS = -(-NBLKS // NS)
EBS = KGS * G
NROWS = N // NS

E4 = E // 4
N2 = N // 2
BLE = 2000
BLN = 1000

def _mesh():
    return plsc.VectorSubcoreMesh(
        core_axis_name="c", subcore_axis_name="s", num_cores=NC, num_subcores=NS
    )


def _gather_body(xpad, recv2d, send2d, r_out, s_out, idx_v, rows_v, sem):
    c = lax.axis_index("c")
    s = lax.axis_index("s")
    wid = s * NC + c

    def one(idx_hbm, out_hbm, b):
        pltpu.sync_copy(idx_hbm.at[pl.ds(b * KG, KG), :], idx_v)
        descs = []
        for j in range(KG):
            descs.append(
                pltpu.async_copy(
                    xpad.at[idx_v.at[j]], rows_v.at[pl.ds(j * G, G), :], sem
                )
            )
        for d in descs:
            d.wait()
        pltpu.sync_copy(rows_v, out_hbm.at[pl.ds(b * EB, EB), :])

    def body(k, carry):
        b = wid + NW * k

        @pl.when(b < NBLK)
        def _():
            one(recv2d, r_out, b)
            one(send2d, s_out, b)

        return carry

    lax.fori_loop(0, KMAX, body, 0)


@functools.lru_cache(maxsize=None)
def _gather_sc():
    return pl.kernel(
        _gather_body,
        out_type=(
            jax.ShapeDtypeStruct((E, 16), jnp.float32),
            jax.ShapeDtypeStruct((E, 16), jnp.float32),
        ),
        mesh=_mesh(),
        scratch_types=[
            pltpu.VMEM((KG, G), jnp.int32),
            pltpu.VMEM((EB, 16), jnp.float32),
            pltpu.SemaphoreType.DMA,
        ],
        compiler_params=pltpu.CompilerParams(use_tc_tiling_on_sc=False),
    )


def _scatter_body(recv2d, send2d, ue, zeros32, recv_out, sent_out,
                  table, idx_v, vals_v):
    c = lax.axis_index("c")
    s = lax.axis_index("s")
    wid = s * NC + c
    col0 = c * 32

    for idx_hbm, out_hbm in ((recv2d, recv_out), (send2d, sent_out)):
        pltpu.sync_copy(
            zeros32.at[pl.ds(s * NROWS, NROWS), :],
            table.at[pl.ds(s * NROWS, NROWS), :],
        )
        plsc.subcore_barrier()

        def body(k, carry):
            b = s + NS * k

            @pl.when(b < NBLKS)
            def _():
                pltpu.sync_copy(idx_hbm.at[pl.ds(b * KGS, KGS), :], idx_v)
                pltpu.sync_copy(
                    ue.at[pl.ds(b * EBS, EBS), pl.ds(col0, 32)], vals_v
                )
                for j in range(KGS):
                    pltpu.sync_copy(
                        vals_v.at[pl.ds(j * G, G), :],
                        table.at[idx_v.at[j]],
                        add=True,
                    )

            return carry

        lax.fori_loop(0, KMAXS, body, 0)
        plsc.subcore_barrier()
        pltpu.sync_copy(
            table.at[pl.ds(s * NROWS, NROWS), :],
            out_hbm.at[pl.ds(s * NROWS, NROWS), pl.ds(col0, 32)],
        )
        plsc.subcore_barrier()


@functools.lru_cache(maxsize=None)
def _scatter_sc():
    return pl.kernel(
        _scatter_body,
        out_type=(
            jax.ShapeDtypeStruct((N, LAT), jnp.float32),
            jax.ShapeDtypeStruct((N, LAT), jnp.float32),
        ),
        mesh=_mesh(),
        scratch_types=[
            pltpu.VMEM_SHARED((N, 32), jnp.float32),
            pltpu.VMEM((KGS, G), jnp.int32),
            pltpu.VMEM((EBS, 32), jnp.float32),
        ],
        compiler_params=pltpu.CompilerParams(use_tc_tiling_on_sc=False),
    )


def _edge_body(eap, rp, sp, w1, b1, w2, b2, out, agg):
    i = pl.program_id(0)
    xin = jnp.concatenate([eap[...], rp[...], sp[...]], axis=1)
    h = jnp.dot(xin, w1[...], preferred_element_type=jnp.float32) + b1[...]
    h = jnp.maximum(h, 0.0)
    o = jnp.dot(h, w2[...], preferred_element_type=jnp.float32) + b2[...]
    out[...] = o

    @pl.when(i == 0)
    def _():
        agg[...] = jnp.zeros_like(agg)

    agg[...] += jnp.sum(o, axis=0, keepdims=True)


def _edge_mlp(eap, rp, sp, w1, b1, w2, b2):
    nsteps = E4 // BLE
    full = lambda shape: pl.BlockSpec(shape, lambda i: (0, 0))
    return pl.pallas_call(
        _edge_body,
        grid=(nsteps,),
        in_specs=[
            pl.BlockSpec((BLE, 32), lambda i: (i, 0)),
            pl.BlockSpec((BLE, 64), lambda i: (i, 0)),
            pl.BlockSpec((BLE, 64), lambda i: (i, 0)),
            full((160, 256)),
            full((1, 256)),
            full((256, 256)),
            full((1, 256)),
        ],
        out_specs=[
            pl.BlockSpec((BLE, 256), lambda i: (i, 0)),
            full((1, 256)),
        ],
        out_shape=[
            jax.ShapeDtypeStruct((E4, 256), jnp.float32),
            jax.ShapeDtypeStruct((1, 256), jnp.float32),
        ],
    )(eap, rp, sp, w1, b1, w2, b2)


def _node_body(rv, sv, xv, w1, b1, w2, b2, eagg, wge, wgn, bg1e, wg2, bg2,
               out, gout, nacc):
    i = pl.program_id(0)
    xin = jnp.concatenate([rv[...], sv[...], xv[...]], axis=1)
    h = jnp.dot(xin, w1[...], preferred_element_type=jnp.float32) + b1[...]
    h = jnp.maximum(h, 0.0)
    o = jnp.dot(h, w2[...], preferred_element_type=jnp.float32) + b2[...]
    out[...] = o

    @pl.when(i == 0)
    def _():
        nacc[...] = jnp.zeros_like(nacc)

    nacc[...] += jnp.sum(o, axis=0, keepdims=True)

    @pl.when(i == pl.num_programs(0) - 1)
    def _():
        hg = (
            jnp.dot(eagg[...], wge[...], preferred_element_type=jnp.float32)
            + jnp.dot(nacc[...], wgn[...], preferred_element_type=jnp.float32)
            + bg1e[...]
        )
        hg = jnp.maximum(hg, 0.0)
        gout[...] = (
            jnp.dot(hg, wg2[...], preferred_element_type=jnp.float32) + bg2[...]
        )


def _node_mlp(rv, sv, xv, w1, b1, w2, b2, eagg, wge, wgn, bg1e, wg2, bg2):
    nsteps = N2 // BLN
    full = lambda shape: pl.BlockSpec(shape, lambda i: (0, 0))
    return pl.pallas_call(
        _node_body,
        grid=(nsteps,),
        in_specs=[
            pl.BlockSpec((BLN, 128), lambda i: (i, 0)),
            pl.BlockSpec((BLN, 128), lambda i: (i, 0)),
            pl.BlockSpec((BLN, 16), lambda i: (i, 0)),
            full((272, 128)),
            full((1, 128)),
            full((128, 128)),
            full((1, 128)),
            full((1, 256)),
            full((256, LAT)),
            full((128, LAT)),
            full((1, LAT)),
            full((LAT, LAT)),
            full((1, LAT)),
        ],
        out_specs=[
            pl.BlockSpec((BLN, 128), lambda i: (i, 0)),
            full((1, LAT)),
        ],
        out_shape=[
            jax.ShapeDtypeStruct((N2, 128), jnp.float32),
            jax.ShapeDtypeStruct((1, LAT), jnp.float32),
        ],
        scratch_shapes=[pltpu.VMEM((1, 128), jnp.float32)],
    )(rv, sv, xv, w1, b1, w2, b2, eagg, wge, wgn, bg1e, wg2, bg2)


def kernel(x, edge_index, edge_attr, u,
           We1, be1, We2, be2,
           Wn1, bn1, Wn2, bn2,
           Wg1, bg1, Wg2, bg2):
    f32 = jnp.float32
    uval = u[0, 0]
    eye4 = jnp.eye(4, dtype=f32)

    send2d = edge_index[0].astype(jnp.int32).reshape(NGRP, G)
    recv2d = edge_index[1].astype(jnp.int32).reshape(NGRP, G)
    xpad = jnp.pad(x, ((0, 0), (0, 11)))
    ea8 = jnp.pad(edge_attr, ((0, 0), (0, 6)))

    r16, s16 = _gather_sc()(xpad, recv2d, send2d)

    Wa8 = jnp.zeros((8, LAT), f32).at[0:2].set(We1[0:2])
    Wr16 = jnp.zeros((16, LAT), f32).at[0:5].set(We1[2:7])
    Ws16 = jnp.zeros((16, LAT), f32).at[0:5].set(We1[7:12])
    b1e = be1 + uval * We1[12]
    W1 = jnp.concatenate(
        [jnp.kron(eye4, Wa8), jnp.kron(eye4, Wr16), jnp.kron(eye4, Ws16)], axis=0
    )
    b1t = jnp.tile(b1e, 4)[None, :]
    W2 = jnp.kron(eye4, We2)
    b2t = jnp.tile(be2, 4)[None, :]

    uep, eagg = _edge_mlp(
        ea8.reshape(E4, 32), r16.reshape(E4, 64),
        s16.reshape(E4, 64), W1, b1t, W2, b2t,
    )
    ue = uep.reshape(E, LAT)

    zeros32 = jnp.zeros((N, 32), f32)
    recv_agg, sent_agg = _scatter_sc()(recv2d, send2d, ue, zeros32)

    eye2 = jnp.eye(2, dtype=f32)
    Wx8 = jnp.zeros((8, LAT), f32).at[0:5].set(Wn1[128:133])
    b1n = bn1 + uval * Wn1[133]
    W1n = jnp.concatenate(
        [jnp.kron(eye2, Wn1[0:64]), jnp.kron(eye2, Wn1[64:128]),
         jnp.kron(eye2, Wx8)], axis=0
    )
    b1nt = jnp.tile(b1n, 2)[None, :]
    W2n = jnp.kron(eye2, Wn2)
    b2nt = jnp.tile(bn2, 2)[None, :]

    wge = jnp.tile(Wg1[0:64], (4, 1))
    wgn = jnp.tile(Wg1[64:128], (2, 1))
    bg1e = (bg1 + uval * Wg1[128])[None, :]

    xp = jnp.pad(x, ((0, 0), (0, 3))).reshape(N2, 16)
    unp, ug = _node_mlp(
        recv_agg.reshape(N2, 128), sent_agg.reshape(N2, 128), xp,
        W1n, b1nt, W2n, b2nt, eagg, wge, wgn, bg1e, Wg2, bg2[None, :],
    )

    return ue, unp.reshape(N, LAT), ug.reshape(LAT)
